# Initial kernel scaffold; baseline (speedup 1.0000x reference)
import jax
import jax.numpy as jnp
from jax.experimental import pallas as pl


def kernel(x, edge_index, edge_attr, params):
    raise NotImplementedError("write your pallas kernel here")


# speedup vs baseline: 1.1416x; 1.1416x over previous
import functools

import jax
import jax.numpy as jnp
from jax.experimental import pallas as pl
from jax.experimental.pallas import tpu as pltpu

_NODE_VOCABS = (120, 10, 12)
_EDGE_VOCABS = (6, 3)
_MASK_RATE = 0.15


def kernel(x, edge_index, edge_attr, params):
    N = x.shape[0]
    E = edge_attr.shape[0]
    H = params["node_emb"][0].shape[1]

    mkey = jax.random.key(42)
    node_mask = jax.random.uniform(jax.random.fold_in(mkey, 0), (N,)) < _MASK_RATE
    edge_mask = jax.random.uniform(jax.random.fold_in(mkey, 1), (E,)) < _MASK_RATE

    nfill = jnp.array([v - 1 for v in _NODE_VOCABS], dtype=x.dtype)
    efill = jnp.array([v - 1 for v in _EDGE_VOCABS], dtype=edge_attr.dtype)
    xm = jnp.where(node_mask[:, None], nfill[None, :], x)
    eam = jnp.where(edge_mask[:, None], efill[None, :], edge_attr)

    h = jnp.zeros((N, H), jnp.float32)
    for i, t in enumerate(params["node_emb"]):
        h = h + jnp.take(t, xm[:, i], axis=0)

    T0, T1 = params["edge_emb"]
    C = (T0[:, None, :] + T1[None, :, :]).reshape(
        _EDGE_VOCABS[0] * _EDGE_VOCABS[1], H)
    ci = eam[:, 0] * _EDGE_VOCABS[1] + eam[:, 1]

    src, dst = edge_index[0], edge_index[1]
    for layer in params["layers"]:
        msgs = jax.nn.relu(jnp.take(h, src, axis=0) + jnp.take(C, ci, axis=0))
        agg = jax.ops.segment_sum(msgs, dst, num_segments=N)
        z = h + agg
        z = jax.nn.relu(z @ layer["lin1"]["W"] + layer["lin1"]["b"])
        h = z @ layer["lin2"]["W"] + layer["lin2"]["b"]

    Wn = jnp.concatenate([hd["W"] for hd in params["node_heads"]], axis=1)
    bn = jnp.concatenate([hd["b"] for hd in params["node_heads"]])
    node_pred = h @ Wn + bn

    We = jnp.concatenate([hd["W"] for hd in params["edge_heads"]], axis=1)
    be = jnp.concatenate([hd["b"] for hd in params["edge_heads"]])
    Q = h @ We
    edge_pred = jnp.take(Q, src, axis=0) + jnp.take(Q, dst, axis=0) + be

    return node_pred, edge_pred, node_mask, edge_mask


# speedup vs baseline: 1.3899x; 1.2175x over previous
import functools

import jax
import jax.numpy as jnp
from jax import lax
from jax.experimental import pallas as pl
from jax.experimental.pallas import tpu as pltpu
from jax.experimental.pallas import tpu_sc as plsc

_NODE_VOCABS = (120, 10, 12)
_EDGE_VOCABS = (6, 3)
_MASK_RATE = 0.15

_NC = 2
_NS = 16
_L = 16

_CH = 128
_HH = 32


def _ceil_to(x, m):
    return (x + m - 1) // m * m


def _msg_agg_kernel(NP, EP, per_sub, nchunk, rows_per_sub):
    mesh = plsc.VectorSubcoreMesh(core_axis_name="c", subcore_axis_name="s")

    def run_half(h, C, agg_out, src_r, dst_r, ci_r, s,
                 sidx, cidx, didx, rows, crows, aggs, sem1, sem2, zer):
        pltpu.sync_copy(zer, aggs.at[pl.ds(s * rows_per_sub, rows_per_sub)])
        plsc.subcore_barrier()

        base0 = s * per_sub

        def chunk(k, carry):
            b = base0 + k * _CH
            pltpu.sync_copy(src_r.at[pl.ds(b, _CH)], sidx)
            pltpu.sync_copy(ci_r.at[pl.ds(b, _CH)], cidx)
            g1 = pltpu.async_copy(h.at[sidx], rows, sem1)
            g2 = pltpu.async_copy(C.at[cidx], crows, sem2)
            pltpu.sync_copy(dst_r.at[pl.ds(b, _CH)], didx)
            g1.wait()
            g2.wait()

            def jbody(j, c2):
                a0 = rows[j, pl.ds(0, _L)]
                c0 = crows[j, pl.ds(0, _L)]
                rows[j, pl.ds(0, _L)] = jnp.maximum(a0 + c0, 0.0)
                a1 = rows[j, pl.ds(_L, _L)]
                c1 = crows[j, pl.ds(_L, _L)]
                rows[j, pl.ds(_L, _L)] = jnp.maximum(a1 + c1, 0.0)
                return c2

            lax.fori_loop(0, _CH, jbody, 0)
            pltpu.sync_copy(rows, aggs.at[didx], add=True)
            return carry

        lax.fori_loop(0, nchunk, chunk, 0)
        plsc.subcore_barrier()
        sl = pl.ds(s * rows_per_sub, rows_per_sub)
        pltpu.sync_copy(aggs.at[sl], agg_out.at[sl])

    @functools.partial(
        pl.kernel,
        out_type=(
            jax.ShapeDtypeStruct((NP, _HH), jnp.float32),
            jax.ShapeDtypeStruct((NP, _HH), jnp.float32),
        ),
        mesh=mesh,
        scratch_types=(
            pltpu.VMEM((_CH,), jnp.int32),
            pltpu.VMEM((_CH,), jnp.int32),
            pltpu.VMEM((_CH,), jnp.int32),
            pltpu.VMEM((_CH, _HH), jnp.float32),
            pltpu.VMEM((_CH, _HH), jnp.float32),
            pltpu.VMEM_SHARED((NP, _HH), jnp.float32),
            pltpu.SemaphoreType.DMA,
            pltpu.SemaphoreType.DMA,
        ),
        compiler_params=pltpu.CompilerParams(use_tc_tiling_on_sc=False),
    )
    def kern(hA, hB, CA, CB, src_r, dst_r, ci_r, zer,
             aggA, aggB,
             sidx, cidx, didx, rows, crows, aggs, sem1, sem2):
        c = lax.axis_index("c")
        s = lax.axis_index("s")

        @pl.when(c == 0)
        def _():
            run_half(hA, CA, aggA, src_r, dst_r, ci_r, s,
                     sidx, cidx, didx, rows, crows, aggs, sem1, sem2, zer)

        @pl.when(c == 1)
        def _():
            run_half(hB, CB, aggB, src_r, dst_r, ci_r, s,
                     sidx, cidx, didx, rows, crows, aggs, sem1, sem2, zer)

    return kern


def kernel(x, edge_index, edge_attr, params):
    N = x.shape[0]
    E = edge_attr.shape[0]
    H = params["node_emb"][0].shape[1]

    mkey = jax.random.key(42)
    node_mask = jax.random.uniform(jax.random.fold_in(mkey, 0), (N,)) < _MASK_RATE
    edge_mask = jax.random.uniform(jax.random.fold_in(mkey, 1), (E,)) < _MASK_RATE

    nfill = jnp.array([v - 1 for v in _NODE_VOCABS], dtype=x.dtype)
    efill = jnp.array([v - 1 for v in _EDGE_VOCABS], dtype=edge_attr.dtype)
    xm = jnp.where(node_mask[:, None], nfill[None, :], x)
    eam = jnp.where(edge_mask[:, None], efill[None, :], edge_attr)

    h = jnp.zeros((N, H), jnp.float32)
    for i, t in enumerate(params["node_emb"]):
        h = h + jnp.take(t, xm[:, i], axis=0)

    T0, T1 = params["edge_emb"]
    C = (T0[:, None, :] + T1[None, :, :]).reshape(
        _EDGE_VOCABS[0] * _EDGE_VOCABS[1], H)
    ci = eam[:, 0] * _EDGE_VOCABS[1] + eam[:, 1]

    per_sub = _ceil_to(-(-E // _NS), _CH)
    EP = per_sub * _NS
    nchunk = per_sub // _CH
    rows_per_sub = _ceil_to(-(-(N + 1) // _NS), 8)
    NP = rows_per_sub * _NS

    src = edge_index[0]
    dst = edge_index[1]
    pad_e = EP - E
    src_p = jnp.pad(src, (0, pad_e))
    dst_p = jnp.pad(dst, (0, pad_e), constant_values=N)
    ci_p = jnp.pad(ci, (0, pad_e))
    zer = jnp.zeros((rows_per_sub, _HH), jnp.float32)
    CA, CB = C[:, :_HH], C[:, _HH:]

    sc_msg_agg = _msg_agg_kernel(NP, EP, per_sub, nchunk, rows_per_sub)

    for layer in params["layers"]:
        hp = jnp.pad(h, ((0, NP - N), (0, 0)))
        aggA, aggB = sc_msg_agg(hp[:, :_HH], hp[:, _HH:], CA, CB,
                                src_p, dst_p, ci_p, zer)
        agg = jnp.concatenate([aggA[:N], aggB[:N]], axis=1)
        z = h + agg
        z = jax.nn.relu(z @ layer["lin1"]["W"] + layer["lin1"]["b"])
        h = z @ layer["lin2"]["W"] + layer["lin2"]["b"]

    Wn = jnp.concatenate([hd["W"] for hd in params["node_heads"]], axis=1)
    bn = jnp.concatenate([hd["b"] for hd in params["node_heads"]])
    node_pred = h @ Wn + bn

    We = jnp.concatenate([hd["W"] for hd in params["edge_heads"]], axis=1)
    be = jnp.concatenate([hd["b"] for hd in params["edge_heads"]])
    Q = h @ We
    edge_pred = jnp.take(Q, src, axis=0) + jnp.take(Q, dst, axis=0) + be

    return node_pred, edge_pred, node_mask, edge_mask
